# Initial kernel scaffold; baseline (speedup 1.0000x reference)
import jax
import jax.numpy as jnp
from jax.experimental import pallas as pl


def kernel(means3D, rays_o, rays_d, rotations, scales, colors, opacities, K, R, t):
    raise NotImplementedError("write your pallas kernel here")


# speedup vs baseline: 8.2204x; 8.2204x over previous
import functools

import numpy as np
import jax
import jax.numpy as jnp
from jax.experimental import pallas as pl
from jax.experimental.pallas import tpu as pltpu

_N = 256
_E = 2 * _N

_STAGES = [(1 << j, 1 << k)
           for k in range(1, _E.bit_length())
           for j in range(k - 1, -1, -1)]
_SS = np.array([s for s, _ in _STAGES], dtype=np.int32)
_KK = np.array([k for _, k in _STAGES], dtype=np.int32)


def _renderer_kernel(rays_ref, par_ref, out_ref):
    B = rays_ref.shape[1]
    rows = jax.lax.broadcasted_iota(jnp.int32, (_E, 1), 0)

    o = [rays_ref[i:i + 1, :] for i in range(3)]
    d = [rays_ref[3 + i:4 + i, :] for i in range(3)]
    dninv = 1.0 / jnp.sqrt(d[0] * d[0] + d[1] * d[1] + d[2] * d[2])
    d = [di * dninv for di in d]

    mean = [par_ref[:, c:c + 1] for c in range(0, 3)]
    scl = [par_ref[:, 12 + c:13 + c] for c in range(3)]
    A = [[par_ref[:, 3 + 3 * j + x:4 + 3 * j + x] / scl[x]
          for j in range(3)] for x in range(3)]
    w = [-(A[x][0] * mean[0] + A[x][1] * mean[1] + A[x][2] * mean[2])
         for x in range(3)]
    op = par_ref[:, 18:19]
    cop = [par_ref[:, 15 + c:16 + c] * op for c in range(3)]

    u = [A[x][0] * o[0] + A[x][1] * o[1] + A[x][2] * o[2] + w[x]
         for x in range(3)]
    v = [A[x][0] * d[0] + A[x][1] * d[1] + A[x][2] * d[2] for x in range(3)]
    qa = v[0] * v[0] + v[1] * v[1] + v[2] * v[2]
    qb = 2.0 * (u[0] * v[0] + u[1] * v[1] + u[2] * v[2])
    qc = u[0] * u[0] + u[1] * u[1] + u[2] * u[2] - 1.0
    disc = qb * qb - 4.0 * qa * qc
    valid = disc >= 0.0
    sq = jnp.sqrt(jnp.maximum(disc, 0.0))
    t0 = (-qb - sq) / (2.0 * qa)
    t1 = (-qb + sq) / (2.0 * qa)
    hit0 = (t0 > 0.0) & valid
    hit1 = (t1 > 0.0) & valid
    t0 = jnp.where(hit0, t0, -1.0)
    t1 = jnp.where(hit1, t1, -1.0)

    zero = jnp.zeros((), jnp.float32)
    tcat = jnp.concatenate([t0, t1], axis=0)
    dd = jnp.concatenate([jnp.where(hit0, op, zero),
                          -jnp.where(hit1, op, zero)], axis=0)
    cd = [jnp.concatenate([jnp.where(hit0, cop[c], zero),
                           jnp.where(hit1, cop[c], zero)], axis=0)
          for c in range(3)]
    P = jnp.concatenate([dd] + cd, axis=1)

    def make_stage(k, kk):
        def stage(i, carry):
            t, p = carry
            s = jnp.int32(1) << (k - 1 - i)
            is_lo = (rows & s) == 0
            desc = (rows & kk) != 0
            want_min = jnp.logical_xor(is_lo, desc)
            t_dn = pltpu.roll(t, _E - s, axis=0)
            t_up = pltpu.roll(t, s, axis=0)
            pk = jnp.where(is_lo, t_dn, t_up)
            take = ((pk < t) == want_min) & (pk != t)
            t_new = jnp.where(take, pk, t)
            p_dn = pltpu.roll(p, _E - s, axis=0)
            p_up = pltpu.roll(p, s, axis=0)
            pp = jnp.where(is_lo, p_dn, p_up)
            take4 = jnp.concatenate([take, take, take, take], axis=1)
            p_new = jnp.where(take4, pp, p)
            return t_new, p_new
        return stage

    carry = (tcat, P)
    for k in range(1, _E.bit_length()):
        carry = jax.lax.fori_loop(0, k, make_stage(k, 1 << k), carry)
    t_s, P_s = carry

    def csum(n_steps, x):
        def step(i, acc):
            s = jnp.int32(1) << i
            sh = pltpu.roll(acc, s, axis=0)
            return acc + jnp.where(rows >= s, sh, zero)
        return jax.lax.fori_loop(0, n_steps, step, x)

    CS = csum(9, P_s)
    D = CS[:, :B]
    t_next = jnp.concatenate([t_s[1:], t_s[_E - 1:]], axis=0)
    x = (t_next - t_s) * D
    S = csum(9, x)
    Aexp = S + jnp.where(rows > 0, x, zero)
    wgt = jnp.where(rows < _E - 1, jnp.exp(-Aexp), zero)
    wc = wgt / jnp.maximum(D, 1e-6)
    outs = [jnp.sum(wc * CS[:, (c + 1) * B:(c + 2) * B], axis=0, keepdims=True)
            for c in range(3)]
    out_ref[:, :] = jnp.concatenate(outs, axis=0)


def _render(rays, params, M, B):
    return pl.pallas_call(
        _renderer_kernel,
        grid=(M // B,),
        in_specs=[
            pl.BlockSpec((6, B), lambda i: (0, i)),
            pl.BlockSpec((_N, 19), lambda i: (0, 0)),
        ],
        out_specs=pl.BlockSpec((3, B), lambda i: (0, i)),
        out_shape=jax.ShapeDtypeStruct((3, M), jnp.float32),
    )(rays, params)


def kernel(means3D, rays_o, rays_d, rotations, scales, colors, opacities,
           K, R, t):
    M = rays_o.shape[0]
    B = 128
    rays = jnp.concatenate([rays_o.T, rays_d.T], axis=0)
    params = jnp.concatenate(
        [means3D, rotations.reshape(_N, 9), scales, colors, opacities],
        axis=1)
    out = _render(rays, params, M, B)
    return out.T


# speedup vs baseline: 17.0350x; 2.0723x over previous
import functools

import numpy as np
import jax
import jax.numpy as jnp
from jax.experimental import pallas as pl
from jax.experimental.pallas import tpu as pltpu

_N = 256
_E = 2 * _N

_STAGES = [(1 << j, 1 << k)
           for k in range(1, _E.bit_length())
           for j in range(k - 1, -1, -1)]
_SS = np.array([s for s, _ in _STAGES], dtype=np.int32)
_KK = np.array([k for _, k in _STAGES], dtype=np.int32)


def _renderer_kernel(rays_ref, par_ref, out_ref):
    B = rays_ref.shape[1]
    rows = jax.lax.broadcasted_iota(jnp.int32, (_E, 1), 0)

    o = [rays_ref[i:i + 1, :] for i in range(3)]
    d = [rays_ref[3 + i:4 + i, :] for i in range(3)]
    dninv = 1.0 / jnp.sqrt(d[0] * d[0] + d[1] * d[1] + d[2] * d[2])
    d = [di * dninv for di in d]

    mean = [par_ref[:, c:c + 1] for c in range(0, 3)]
    scl = [par_ref[:, 12 + c:13 + c] for c in range(3)]
    A = [[par_ref[:, 3 + 3 * j + x:4 + 3 * j + x] / scl[x]
          for j in range(3)] for x in range(3)]
    w = [-(A[x][0] * mean[0] + A[x][1] * mean[1] + A[x][2] * mean[2])
         for x in range(3)]
    op = par_ref[:, 18:19]
    cop = [par_ref[:, 15 + c:16 + c] * op for c in range(3)]

    u = [A[x][0] * o[0] + A[x][1] * o[1] + A[x][2] * o[2] + w[x]
         for x in range(3)]
    v = [A[x][0] * d[0] + A[x][1] * d[1] + A[x][2] * d[2] for x in range(3)]
    qa = v[0] * v[0] + v[1] * v[1] + v[2] * v[2]
    qb = 2.0 * (u[0] * v[0] + u[1] * v[1] + u[2] * v[2])
    qc = u[0] * u[0] + u[1] * u[1] + u[2] * u[2] - 1.0
    disc = qb * qb - 4.0 * qa * qc
    valid = disc >= 0.0
    sq = jnp.sqrt(jnp.maximum(disc, 0.0))
    t0 = (-qb - sq) / (2.0 * qa)
    t1 = (-qb + sq) / (2.0 * qa)
    hit0 = (t0 > 0.0) & valid
    hit1 = (t1 > 0.0) & valid
    t0 = jnp.where(hit0, t0, -1.0)
    t1 = jnp.where(hit1, t1, -1.0)

    zero = jnp.zeros((), jnp.float32)
    tcat = jnp.concatenate([t0, t1], axis=0)
    dd = jnp.concatenate([jnp.where(hit0, op, zero),
                          -jnp.where(hit1, op, zero)], axis=0)
    qc = [jnp.round(cop[c] * 1023.0).astype(jnp.int32) for c in range(3)]
    packi = (qc[0] << 20) | (qc[1] << 10) | qc[2]
    packf = jax.lax.bitcast_convert_type(packi, jnp.float32)
    pcd = jnp.concatenate([jnp.where(hit0, packf, zero),
                           jnp.where(hit1, packf, zero)], axis=0)
    arr = jnp.concatenate([tcat, dd, pcd], axis=1)

    def make_stage(k, kk):
        def stage(i, a):
            s = jnp.int32(1) << (k - 1 - i)
            is_lo = (rows & s) == 0
            desc = (rows & kk) != 0
            want_min = jnp.logical_xor(is_lo, desc)
            a_dn = pltpu.roll(a, _E - s, axis=0)
            a_up = pltpu.roll(a, s, axis=0)
            ap = jnp.where(is_lo, a_dn, a_up)
            t = a[:, :B]
            pk = ap[:, :B]
            take = ((pk < t) == want_min) & (pk != t)
            take3 = jnp.concatenate([take, take, take], axis=1)
            return jnp.where(take3, ap, a)
        return stage

    for k in range(1, _E.bit_length()):
        arr = jax.lax.fori_loop(0, k, make_stage(k, 1 << k), arr)
    t_s = arr[:, :B]
    dd_s = arr[:, B:2 * B]
    pk_s = jax.lax.bitcast_convert_type(arr[:, 2 * B:], jnp.int32)
    inv1023 = jnp.float32(1.0 / 1023.0)
    cd_s = [((pk_s >> 20) & 1023).astype(jnp.float32) * inv1023,
            ((pk_s >> 10) & 1023).astype(jnp.float32) * inv1023,
            (pk_s & 1023).astype(jnp.float32) * inv1023]
    P_s = jnp.concatenate([dd_s] + cd_s, axis=1)

    def csum(n_steps, x):
        def step(i, acc):
            s = jnp.int32(1) << i
            sh = pltpu.roll(acc, s, axis=0)
            return acc + jnp.where(rows >= s, sh, zero)
        return jax.lax.fori_loop(0, n_steps, step, x)

    CS = csum(9, P_s)
    D = CS[:, :B]
    t_next = jnp.concatenate([t_s[1:], t_s[_E - 1:]], axis=0)
    x = (t_next - t_s) * D
    S = csum(9, x)
    Aexp = S + jnp.where(rows > 0, x, zero)
    wgt = jnp.where(rows < _E - 1, jnp.exp(-Aexp), zero)
    wc = wgt / jnp.maximum(D, 1e-6)
    outs = [jnp.sum(wc * CS[:, (c + 1) * B:(c + 2) * B], axis=0, keepdims=True)
            for c in range(3)]
    out_ref[:, :] = jnp.concatenate(outs, axis=0)


def _render(rays, params, M, B):
    return pl.pallas_call(
        _renderer_kernel,
        grid=(M // B,),
        in_specs=[
            pl.BlockSpec((6, B), lambda i: (0, i)),
            pl.BlockSpec((_N, 19), lambda i: (0, 0)),
        ],
        out_specs=pl.BlockSpec((3, B), lambda i: (0, i)),
        out_shape=jax.ShapeDtypeStruct((3, M), jnp.float32),
    )(rays, params)


def kernel(means3D, rays_o, rays_d, rotations, scales, colors, opacities,
           K, R, t):
    M = rays_o.shape[0]
    B = 128
    rays = jnp.concatenate([rays_o.T, rays_d.T], axis=0)
    params = jnp.concatenate(
        [means3D, rotations.reshape(_N, 9), scales, colors, opacities],
        axis=1)
    out = _render(rays, params, M, B)
    return out.T


# speedup vs baseline: 45.8101x; 2.6892x over previous
import functools

import numpy as np
import jax
import jax.numpy as jnp
from jax.experimental import pallas as pl
from jax.experimental.pallas import tpu as pltpu

_N = 256
_E = 2 * _N

_STAGES = [(1 << j, 1 << k)
           for k in range(1, _E.bit_length())
           for j in range(k - 1, -1, -1)]
_SS = np.array([s for s, _ in _STAGES], dtype=np.int32)
_KK = np.array([k for _, k in _STAGES], dtype=np.int32)


def _renderer_kernel(rays_ref, par_ref, out_ref):
    B = rays_ref.shape[1]
    rows = jax.lax.broadcasted_iota(jnp.int32, (_E, 1), 0)

    o = [rays_ref[i:i + 1, :] for i in range(3)]
    d = [rays_ref[3 + i:4 + i, :] for i in range(3)]
    dninv = 1.0 / jnp.sqrt(d[0] * d[0] + d[1] * d[1] + d[2] * d[2])
    d = [di * dninv for di in d]

    mean = [par_ref[:, c:c + 1] for c in range(0, 3)]
    scl = [par_ref[:, 12 + c:13 + c] for c in range(3)]
    A = [[par_ref[:, 3 + 3 * j + x:4 + 3 * j + x] / scl[x]
          for j in range(3)] for x in range(3)]
    w = [-(A[x][0] * mean[0] + A[x][1] * mean[1] + A[x][2] * mean[2])
         for x in range(3)]
    op = par_ref[:, 18:19]
    cop = [par_ref[:, 15 + c:16 + c] * op for c in range(3)]

    u = [A[x][0] * o[0] + A[x][1] * o[1] + A[x][2] * o[2] + w[x]
         for x in range(3)]
    v = [A[x][0] * d[0] + A[x][1] * d[1] + A[x][2] * d[2] for x in range(3)]
    qa = v[0] * v[0] + v[1] * v[1] + v[2] * v[2]
    qb = 2.0 * (u[0] * v[0] + u[1] * v[1] + u[2] * v[2])
    qc = u[0] * u[0] + u[1] * u[1] + u[2] * u[2] - 1.0
    disc = qb * qb - 4.0 * qa * qc
    valid = disc >= 0.0
    sq = jnp.sqrt(jnp.maximum(disc, 0.0))
    t0 = (-qb - sq) / (2.0 * qa)
    t1 = (-qb + sq) / (2.0 * qa)
    hit0 = (t0 > 0.0) & valid
    hit1 = (t1 > 0.0) & valid
    t0 = jnp.where(hit0, t0, -1.0)
    t1 = jnp.where(hit1, t1, -1.0)

    zero = jnp.zeros((), jnp.float32)
    tcat = jnp.concatenate([t0, t1], axis=0)
    dd = jnp.concatenate([jnp.where(hit0, op, zero),
                          -jnp.where(hit1, op, zero)], axis=0)
    qc = [jnp.round(cop[c] * 1023.0).astype(jnp.int32) for c in range(3)]
    packi = (qc[0] << 20) | (qc[1] << 10) | qc[2]
    packf = jax.lax.bitcast_convert_type(packi, jnp.float32)
    pcd = jnp.concatenate([jnp.where(hit0, packf, zero),
                           jnp.where(hit1, packf, zero)], axis=0)
    arr = jnp.concatenate([tcat, dd, pcd], axis=1)

    def stage(a, s, kk):
        is_lo = (rows & s) == 0
        desc = (rows & kk) != 0
        want_min = jnp.logical_xor(is_lo, desc)
        a_dn = pltpu.roll(a, _E - s, axis=0)
        a_up = pltpu.roll(a, s, axis=0)
        ap = jnp.where(is_lo, a_dn, a_up)
        t = a[:, :B]
        pk = ap[:, :B]
        take = ((pk < t) == want_min) & (pk != t)
        take3 = jnp.concatenate([take, take, take], axis=1)
        return jnp.where(take3, ap, a)

    for k in range(1, _E.bit_length()):
        for j in range(k - 1, -1, -1):
            arr = stage(arr, 1 << j, 1 << k)
    t_s = arr[:, :B]
    dd_s = arr[:, B:2 * B]
    pk_s = jax.lax.bitcast_convert_type(arr[:, 2 * B:], jnp.int32)
    inv1023 = jnp.float32(1.0 / 1023.0)
    cd_s = [((pk_s >> 20) & 1023).astype(jnp.float32) * inv1023,
            ((pk_s >> 10) & 1023).astype(jnp.float32) * inv1023,
            (pk_s & 1023).astype(jnp.float32) * inv1023]
    P_s = jnp.concatenate([dd_s] + cd_s, axis=1)

    def csum(n_steps, x):
        def step(i, acc):
            s = jnp.int32(1) << i
            sh = pltpu.roll(acc, s, axis=0)
            return acc + jnp.where(rows >= s, sh, zero)
        return jax.lax.fori_loop(0, n_steps, step, x)

    CS = csum(9, P_s)
    D = CS[:, :B]
    t_next = jnp.concatenate([t_s[1:], t_s[_E - 1:]], axis=0)
    x = (t_next - t_s) * D
    S = csum(9, x)
    Aexp = S + jnp.where(rows > 0, x, zero)
    wgt = jnp.where(rows < _E - 1, jnp.exp(-Aexp), zero)
    wc = wgt / jnp.maximum(D, 1e-6)
    outs = [jnp.sum(wc * CS[:, (c + 1) * B:(c + 2) * B], axis=0, keepdims=True)
            for c in range(3)]
    out_ref[:, :] = jnp.concatenate(outs, axis=0)


def _render(rays, params, M, B):
    return pl.pallas_call(
        _renderer_kernel,
        grid=(M // B,),
        in_specs=[
            pl.BlockSpec((6, B), lambda i: (0, i)),
            pl.BlockSpec((_N, 19), lambda i: (0, 0)),
        ],
        out_specs=pl.BlockSpec((3, B), lambda i: (0, i)),
        out_shape=jax.ShapeDtypeStruct((3, M), jnp.float32),
    )(rays, params)


def kernel(means3D, rays_o, rays_d, rotations, scales, colors, opacities,
           K, R, t):
    M = rays_o.shape[0]
    B = 128
    rays = jnp.concatenate([rays_o.T, rays_d.T], axis=0)
    params = jnp.concatenate(
        [means3D, rotations.reshape(_N, 9), scales, colors, opacities],
        axis=1)
    out = _render(rays, params, M, B)
    return out.T


# speedup vs baseline: 66.5508x; 1.4528x over previous
import functools

import numpy as np
import jax
import jax.numpy as jnp
from jax.experimental import pallas as pl
from jax.experimental.pallas import tpu as pltpu

_N = 256
_E = 2 * _N

_STAGES = [(1 << j, 1 << k)
           for k in range(1, _E.bit_length())
           for j in range(k - 1, -1, -1)]
_SS = np.array([s for s, _ in _STAGES], dtype=np.int32)
_KK = np.array([k for _, k in _STAGES], dtype=np.int32)


def _renderer_kernel(rays_ref, par_ref, out_ref):
    B = rays_ref.shape[1]
    rows = jax.lax.broadcasted_iota(jnp.int32, (_E, 1), 0)

    o = [rays_ref[i:i + 1, :] for i in range(3)]
    d = [rays_ref[3 + i:4 + i, :] for i in range(3)]
    dninv = 1.0 / jnp.sqrt(d[0] * d[0] + d[1] * d[1] + d[2] * d[2])
    d = [di * dninv for di in d]

    mean = [par_ref[:, c:c + 1] for c in range(0, 3)]
    scl = [par_ref[:, 12 + c:13 + c] for c in range(3)]
    A = [[par_ref[:, 3 + 3 * j + x:4 + 3 * j + x] / scl[x]
          for j in range(3)] for x in range(3)]
    w = [-(A[x][0] * mean[0] + A[x][1] * mean[1] + A[x][2] * mean[2])
         for x in range(3)]
    op = par_ref[:, 18:19]
    cop = [par_ref[:, 15 + c:16 + c] * op for c in range(3)]

    u = [A[x][0] * o[0] + A[x][1] * o[1] + A[x][2] * o[2] + w[x]
         for x in range(3)]
    v = [A[x][0] * d[0] + A[x][1] * d[1] + A[x][2] * d[2] for x in range(3)]
    qa = v[0] * v[0] + v[1] * v[1] + v[2] * v[2]
    qb = 2.0 * (u[0] * v[0] + u[1] * v[1] + u[2] * v[2])
    qc = u[0] * u[0] + u[1] * u[1] + u[2] * u[2] - 1.0
    disc = qb * qb - 4.0 * qa * qc
    valid = disc >= 0.0
    sq = jnp.sqrt(jnp.maximum(disc, 0.0))
    t0 = (-qb - sq) / (2.0 * qa)
    t1 = (-qb + sq) / (2.0 * qa)
    hit0 = (t0 > 0.0) & valid
    hit1 = (t1 > 0.0) & valid
    t0 = jnp.where(hit0, t0, -1.0)
    t1 = jnp.where(hit1, t1, -1.0)

    zero = jnp.zeros((), jnp.float32)
    tcat = jnp.concatenate([t0, t1], axis=0)
    dd = jnp.concatenate([jnp.where(hit0, op, zero),
                          -jnp.where(hit1, op, zero)], axis=0)
    qc = [jnp.round(cop[c] * 1023.0).astype(jnp.int32) for c in range(3)]
    packi = (qc[0] << 20) | (qc[1] << 10) | qc[2]
    packf = jax.lax.bitcast_convert_type(packi, jnp.float32)
    pcd = jnp.concatenate([jnp.where(hit0, packf, zero),
                           jnp.where(hit1, packf, zero)], axis=0)
    arr = jnp.concatenate([tcat, dd, pcd], axis=1)

    def stage(a, s, kk):
        is_lo = (rows & s) == 0
        desc = (rows & kk) != 0
        want_min = jnp.logical_xor(is_lo, desc)
        a_dn = pltpu.roll(a, _E - s, axis=0)
        a_up = pltpu.roll(a, s, axis=0)
        ap = jnp.where(is_lo, a_dn, a_up)
        t = a[:, :B]
        pk = ap[:, :B]
        take = ((pk < t) == want_min) & (pk != t)
        take3 = jnp.concatenate([take, take, take], axis=1)
        return jnp.where(take3, ap, a)

    for k in range(1, _E.bit_length()):
        for j in range(k - 1, -1, -1):
            arr = stage(arr, 1 << j, 1 << k)
    t_s = arr[:, :B]
    dd_s = arr[:, B:2 * B]
    pk_s = jax.lax.bitcast_convert_type(arr[:, 2 * B:], jnp.int32)
    inv1023 = jnp.float32(1.0 / 1023.0)
    cd_s = [((pk_s >> 20) & 1023).astype(jnp.float32) * inv1023,
            ((pk_s >> 10) & 1023).astype(jnp.float32) * inv1023,
            (pk_s & 1023).astype(jnp.float32) * inv1023]
    P_s = jnp.concatenate([dd_s] + cd_s, axis=1)

    def csum(n_steps, x):
        for i in range(n_steps):
            s = 1 << i
            sh = pltpu.roll(x, s, axis=0)
            x = x + jnp.where(rows >= s, sh, zero)
        return x

    CS = csum(9, P_s)
    D = CS[:, :B]
    t_next = jnp.concatenate([t_s[1:], t_s[_E - 1:]], axis=0)
    x = (t_next - t_s) * D
    S = csum(9, x)
    Aexp = S + jnp.where(rows > 0, x, zero)
    wgt = jnp.where(rows < _E - 1, jnp.exp(-Aexp), zero)
    wc = wgt / jnp.maximum(D, 1e-6)
    outs = [jnp.sum(wc * CS[:, (c + 1) * B:(c + 2) * B], axis=0, keepdims=True)
            for c in range(3)]
    out_ref[:, :] = jnp.concatenate(outs, axis=0)


def _render(rays, params, M, B):
    return pl.pallas_call(
        _renderer_kernel,
        grid=(M // B,),
        in_specs=[
            pl.BlockSpec((6, B), lambda i: (0, i)),
            pl.BlockSpec((_N, 19), lambda i: (0, 0)),
        ],
        out_specs=pl.BlockSpec((3, B), lambda i: (0, i)),
        out_shape=jax.ShapeDtypeStruct((3, M), jnp.float32),
    )(rays, params)


def kernel(means3D, rays_o, rays_d, rotations, scales, colors, opacities,
           K, R, t):
    M = rays_o.shape[0]
    B = 128
    rays = jnp.concatenate([rays_o.T, rays_d.T], axis=0)
    params = jnp.concatenate(
        [means3D, rotations.reshape(_N, 9), scales, colors, opacities],
        axis=1)
    out = _render(rays, params, M, B)
    return out.T


# speedup vs baseline: 66.5555x; 1.0001x over previous
import functools

import numpy as np
import jax
import jax.numpy as jnp
from jax.experimental import pallas as pl
from jax.experimental.pallas import tpu as pltpu

_N = 256
_E = 2 * _N

_STAGES = [(1 << j, 1 << k)
           for k in range(1, _E.bit_length())
           for j in range(k - 1, -1, -1)]
_SS = np.array([s for s, _ in _STAGES], dtype=np.int32)
_KK = np.array([k for _, k in _STAGES], dtype=np.int32)


def _renderer_kernel(rays_ref, par_ref, out_ref):
    B = rays_ref.shape[1]
    rows = jax.lax.broadcasted_iota(jnp.int32, (_E, 1), 0)

    o = [rays_ref[i:i + 1, :] for i in range(3)]
    d = [rays_ref[3 + i:4 + i, :] for i in range(3)]
    dninv = 1.0 / jnp.sqrt(d[0] * d[0] + d[1] * d[1] + d[2] * d[2])
    d = [di * dninv for di in d]

    mean = [par_ref[:, c:c + 1] for c in range(0, 3)]
    scl = [par_ref[:, 12 + c:13 + c] for c in range(3)]
    A = [[par_ref[:, 3 + 3 * j + x:4 + 3 * j + x] / scl[x]
          for j in range(3)] for x in range(3)]
    w = [-(A[x][0] * mean[0] + A[x][1] * mean[1] + A[x][2] * mean[2])
         for x in range(3)]
    op = par_ref[:, 18:19]
    cop = [par_ref[:, 15 + c:16 + c] * op for c in range(3)]

    u = [A[x][0] * o[0] + A[x][1] * o[1] + A[x][2] * o[2] + w[x]
         for x in range(3)]
    v = [A[x][0] * d[0] + A[x][1] * d[1] + A[x][2] * d[2] for x in range(3)]
    qa = v[0] * v[0] + v[1] * v[1] + v[2] * v[2]
    qb = 2.0 * (u[0] * v[0] + u[1] * v[1] + u[2] * v[2])
    qc = u[0] * u[0] + u[1] * u[1] + u[2] * u[2] - 1.0
    disc = qb * qb - 4.0 * qa * qc
    valid = disc >= 0.0
    sq = jnp.sqrt(jnp.maximum(disc, 0.0))
    t0 = (-qb - sq) / (2.0 * qa)
    t1 = (-qb + sq) / (2.0 * qa)
    hit0 = (t0 > 0.0) & valid
    hit1 = (t1 > 0.0) & valid
    t0 = jnp.where(hit0, t0, -1.0)
    t1 = jnp.where(hit1, t1, -1.0)

    zero = jnp.zeros((), jnp.float32)
    tcat = jnp.concatenate([t0, t1], axis=0)
    dd = jnp.concatenate([jnp.where(hit0, op, zero),
                          -jnp.where(hit1, op, zero)], axis=0)
    qc = [jnp.round(cop[c] * 1023.0).astype(jnp.int32) for c in range(3)]
    packi = (qc[0] << 20) | (qc[1] << 10) | qc[2]
    packf = jax.lax.bitcast_convert_type(packi, jnp.float32)
    pcd = jnp.concatenate([jnp.where(hit0, packf, zero),
                           jnp.where(hit1, packf, zero)], axis=0)
    arr = jnp.concatenate([tcat, dd, pcd], axis=1)

    def stage(a, s, kk):
        is_lo = (rows & s) == 0
        desc = (rows & kk) != 0
        want_min = jnp.logical_xor(is_lo, desc)
        a_dn = pltpu.roll(a, _E - s, axis=0)
        a_up = pltpu.roll(a, s, axis=0)
        ap = jnp.where(is_lo, a_dn, a_up)
        t = a[:, :B]
        pk = ap[:, :B]
        take = ((pk < t) == want_min) & (pk != t)
        take3 = jnp.concatenate([take, take, take], axis=1)
        return jnp.where(take3, ap, a)

    for k in range(1, _E.bit_length()):
        for j in range(k - 1, -1, -1):
            arr = stage(arr, 1 << j, 1 << k)
    t_s = arr[:, :B]
    dd_s = arr[:, B:2 * B]
    pk_s = jax.lax.bitcast_convert_type(arr[:, 2 * B:], jnp.int32)
    inv1023 = jnp.float32(1.0 / 1023.0)
    cd_s = [((pk_s >> 20) & 1023).astype(jnp.float32) * inv1023,
            ((pk_s >> 10) & 1023).astype(jnp.float32) * inv1023,
            (pk_s & 1023).astype(jnp.float32) * inv1023]
    P_s = jnp.concatenate([dd_s] + cd_s, axis=1)

    def csum(n_steps, x):
        for i in range(n_steps):
            s = 1 << i
            sh = pltpu.roll(x, s, axis=0)
            x = x + jnp.where(rows >= s, sh, zero)
        return x

    CS = csum(9, P_s)
    D = CS[:, :B]
    t_next = jnp.concatenate([t_s[1:], t_s[_E - 1:]], axis=0)
    x = (t_next - t_s) * D
    S = csum(9, x)
    Aexp = S + jnp.where(rows > 0, x, zero)
    wgt = jnp.where(rows < _E - 1, jnp.exp(-Aexp), zero)
    wc = wgt / jnp.maximum(D, 1e-6)
    outs = [jnp.sum(wc * CS[:, (c + 1) * B:(c + 2) * B], axis=0, keepdims=True)
            for c in range(3)]
    out_ref[:, :] = jnp.concatenate(outs, axis=0)


def _render(rays, params, M, B):
    return pl.pallas_call(
        _renderer_kernel,
        grid=(M // B,),
        in_specs=[
            pl.BlockSpec((6, B), lambda i: (0, i)),
            pl.BlockSpec((_N, 19), lambda i: (0, 0)),
        ],
        out_specs=pl.BlockSpec((3, B), lambda i: (0, i)),
        out_shape=jax.ShapeDtypeStruct((3, M), jnp.float32),
        compiler_params=pltpu.CompilerParams(
            dimension_semantics=("parallel",)),
    )(rays, params)


def kernel(means3D, rays_o, rays_d, rotations, scales, colors, opacities,
           K, R, t):
    M = rays_o.shape[0]
    B = 128
    rays = jnp.concatenate([rays_o.T, rays_d.T], axis=0)
    params = jnp.concatenate(
        [means3D, rotations.reshape(_N, 9), scales, colors, opacities],
        axis=1)
    out = _render(rays, params, M, B)
    return out.T


# speedup vs baseline: 66.6157x; 1.0009x over previous
import functools

import numpy as np
import jax
import jax.numpy as jnp
from jax.experimental import pallas as pl
from jax.experimental.pallas import tpu as pltpu

_N = 256
_E = 2 * _N

_STAGES = [(1 << j, 1 << k)
           for k in range(1, _E.bit_length())
           for j in range(k - 1, -1, -1)]
_SS = np.array([s for s, _ in _STAGES], dtype=np.int32)
_KK = np.array([k for _, k in _STAGES], dtype=np.int32)


def _renderer_kernel(rays_ref, par_ref, out_ref):
    B = rays_ref.shape[1]
    rows = jax.lax.broadcasted_iota(jnp.int32, (_E, 1), 0)

    o = [rays_ref[i:i + 1, :] for i in range(3)]
    d = [rays_ref[3 + i:4 + i, :] for i in range(3)]
    dninv = 1.0 / jnp.sqrt(d[0] * d[0] + d[1] * d[1] + d[2] * d[2])
    d = [di * dninv for di in d]

    mean = [par_ref[:, c:c + 1] for c in range(0, 3)]
    scl = [par_ref[:, 12 + c:13 + c] for c in range(3)]
    A = [[par_ref[:, 3 + 3 * j + x:4 + 3 * j + x] / scl[x]
          for j in range(3)] for x in range(3)]
    w = [-(A[x][0] * mean[0] + A[x][1] * mean[1] + A[x][2] * mean[2])
         for x in range(3)]
    op = par_ref[:, 18:19]
    cop = [par_ref[:, 15 + c:16 + c] * op for c in range(3)]

    u = [A[x][0] * o[0] + A[x][1] * o[1] + A[x][2] * o[2] + w[x]
         for x in range(3)]
    v = [A[x][0] * d[0] + A[x][1] * d[1] + A[x][2] * d[2] for x in range(3)]
    qa = v[0] * v[0] + v[1] * v[1] + v[2] * v[2]
    qb = 2.0 * (u[0] * v[0] + u[1] * v[1] + u[2] * v[2])
    qc = u[0] * u[0] + u[1] * u[1] + u[2] * u[2] - 1.0
    disc = qb * qb - 4.0 * qa * qc
    valid = disc >= 0.0
    sq = jnp.sqrt(jnp.maximum(disc, 0.0))
    t0 = (-qb - sq) / (2.0 * qa)
    t1 = (-qb + sq) / (2.0 * qa)
    hit0 = (t0 > 0.0) & valid
    hit1 = (t1 > 0.0) & valid
    t0 = jnp.where(hit0, t0, -1.0)
    t1 = jnp.where(hit1, t1, -1.0)

    zero = jnp.zeros((), jnp.float32)
    tcat = jnp.concatenate([t0, t1], axis=0)
    dd = jnp.concatenate([jnp.where(hit0, op, zero),
                          -jnp.where(hit1, op, zero)], axis=0)
    qc = [jnp.round(cop[c] * 1023.0).astype(jnp.int32) for c in range(3)]
    packi = (qc[0] << 20) | (qc[1] << 10) | qc[2]
    packf = jax.lax.bitcast_convert_type(packi, jnp.float32)
    pcd = jnp.concatenate([jnp.where(hit0, packf, zero),
                           jnp.where(hit1, packf, zero)], axis=0)
    arr = jnp.concatenate([tcat, dd, pcd], axis=1)

    def stage(a, s, kk):
        is_lo = (rows & s) == 0
        desc = (rows & kk) != 0
        want_min = jnp.logical_xor(is_lo, desc)
        a_dn = pltpu.roll(a, _E - s, axis=0)
        a_up = pltpu.roll(a, s, axis=0)
        ap = jnp.where(is_lo, a_dn, a_up)
        t = a[:, :B]
        pk = ap[:, :B]
        take = ((pk < t) == want_min) & (pk != t)
        take3 = jnp.concatenate([take, take, take], axis=1)
        return jnp.where(take3, ap, a)

    for k in range(1, _E.bit_length()):
        for j in range(k - 1, -1, -1):
            arr = stage(arr, 1 << j, 1 << k)
    t_s = arr[:, :B]
    dd_s = arr[:, B:2 * B]
    pk_s = jax.lax.bitcast_convert_type(arr[:, 2 * B:], jnp.int32)
    inv1023 = jnp.float32(1.0 / 1023.0)
    cd_s = [((pk_s >> 20) & 1023).astype(jnp.float32) * inv1023,
            ((pk_s >> 10) & 1023).astype(jnp.float32) * inv1023,
            (pk_s & 1023).astype(jnp.float32) * inv1023]
    P_s = jnp.concatenate([dd_s] + cd_s, axis=1)

    def csum(n_steps, x):
        for i in range(n_steps):
            s = 1 << i
            sh = pltpu.roll(x, s, axis=0)
            x = x + jnp.where(rows >= s, sh, zero)
        return x

    CS = csum(9, P_s)
    D = CS[:, :B]
    t_next = jnp.concatenate([t_s[1:], t_s[_E - 1:]], axis=0)
    x = (t_next - t_s) * D
    S = csum(9, x)
    Aexp = S + jnp.where(rows > 0, x, zero)
    wgt = jnp.where(rows < _E - 1, jnp.exp(-Aexp), zero)
    wc = wgt / jnp.maximum(D, 1e-6)
    outs = [jnp.sum(wc * CS[:, (c + 1) * B:(c + 2) * B], axis=0, keepdims=True)
            for c in range(3)]
    out_ref[:, :] = jnp.concatenate(outs, axis=0)


def _render(rays, params, M, B):
    return pl.pallas_call(
        _renderer_kernel,
        grid=(M // B,),
        in_specs=[
            pl.BlockSpec((6, B), lambda i: (0, i)),
            pl.BlockSpec((_N, 19), lambda i: (0, 0)),
        ],
        out_specs=pl.BlockSpec((3, B), lambda i: (0, i)),
        out_shape=jax.ShapeDtypeStruct((3, M), jnp.float32),
        compiler_params=pltpu.CompilerParams(
            dimension_semantics=("parallel",)),
    )(rays, params)


def kernel(means3D, rays_o, rays_d, rotations, scales, colors, opacities,
           K, R, t):
    M = rays_o.shape[0]
    B = 256
    rays = jnp.concatenate([rays_o.T, rays_d.T], axis=0)
    params = jnp.concatenate(
        [means3D, rotations.reshape(_N, 9), scales, colors, opacities],
        axis=1)
    out = _render(rays, params, M, B)
    return out.T


# speedup vs baseline: 76.4252x; 1.1473x over previous
import functools

import numpy as np
import jax
import jax.numpy as jnp
from jax.experimental import pallas as pl
from jax.experimental.pallas import tpu as pltpu

_N = 256
_E = 2 * _N

_STAGES = [(1 << j, 1 << k)
           for k in range(1, _E.bit_length())
           for j in range(k - 1, -1, -1)]
_SS = np.array([s for s, _ in _STAGES], dtype=np.int32)
_KK = np.array([k for _, k in _STAGES], dtype=np.int32)


def _renderer_kernel(rays_ref, par_ref, out_ref):
    B = rays_ref.shape[1]
    rows = jax.lax.broadcasted_iota(jnp.int32, (_E, 1), 0)

    o = [rays_ref[i:i + 1, :] for i in range(3)]
    d = [rays_ref[3 + i:4 + i, :] for i in range(3)]
    dninv = 1.0 / jnp.sqrt(d[0] * d[0] + d[1] * d[1] + d[2] * d[2])
    d = [di * dninv for di in d]

    mean = [par_ref[:, c:c + 1] for c in range(0, 3)]
    scl = [par_ref[:, 12 + c:13 + c] for c in range(3)]
    A = [[par_ref[:, 3 + 3 * j + x:4 + 3 * j + x] / scl[x]
          for j in range(3)] for x in range(3)]
    w = [-(A[x][0] * mean[0] + A[x][1] * mean[1] + A[x][2] * mean[2])
         for x in range(3)]
    op = par_ref[:, 18:19]
    cop = [par_ref[:, 15 + c:16 + c] * op for c in range(3)]

    u = [A[x][0] * o[0] + A[x][1] * o[1] + A[x][2] * o[2] + w[x]
         for x in range(3)]
    v = [A[x][0] * d[0] + A[x][1] * d[1] + A[x][2] * d[2] for x in range(3)]
    qa = v[0] * v[0] + v[1] * v[1] + v[2] * v[2]
    qb = 2.0 * (u[0] * v[0] + u[1] * v[1] + u[2] * v[2])
    qc = u[0] * u[0] + u[1] * u[1] + u[2] * u[2] - 1.0
    disc = qb * qb - 4.0 * qa * qc
    valid = disc >= 0.0
    sq = jnp.sqrt(jnp.maximum(disc, 0.0))
    t0 = (-qb - sq) / (2.0 * qa)
    t1 = (-qb + sq) / (2.0 * qa)
    hit0 = (t0 > 0.0) & valid
    hit1 = (t1 > 0.0) & valid
    t0 = jnp.where(hit0, t0, -1.0)
    t1 = jnp.where(hit1, t1, -1.0)

    zero = jnp.zeros((), jnp.float32)
    tcat = jnp.concatenate([t0, t1], axis=0)
    dd = jnp.concatenate([jnp.where(hit0, op, zero),
                          -jnp.where(hit1, op, zero)], axis=0)
    qc = [jnp.round(cop[c] * 1023.0).astype(jnp.int32) for c in range(3)]
    packi = (qc[0] << 20) | (qc[1] << 10) | qc[2]
    packf = jax.lax.bitcast_convert_type(packi, jnp.float32)
    pcd = jnp.concatenate([jnp.where(hit0, packf, zero),
                           jnp.where(hit1, packf, zero)], axis=0)
    arr = jnp.concatenate([tcat, dd, pcd], axis=1)

    sign_bit = jnp.int32(-2147483648)
    i0 = jnp.int32(0)

    def stage(a, s, kk):
        is_lo = (rows & s) == 0
        desc = (rows & kk) != 0
        want_min = jnp.logical_xor(is_lo, desc)
        a_dn = pltpu.roll(a, _E - s, axis=0)
        a_up = pltpu.roll(a, s, axis=0)
        ap = jnp.where(is_lo, a_dn, a_up)
        t = a[:, :B]
        pk = ap[:, :B]
        flip = jnp.where(want_min, i0, sign_bit)
        d = jax.lax.bitcast_convert_type(t - pk, jnp.int32) ^ flip
        take = jax.lax.bitcast_convert_type(d, jnp.float32) > 0.0
        take3 = jnp.concatenate([take, take, take], axis=1)
        return jnp.where(take3, ap, a)

    for k in range(1, _E.bit_length()):
        for j in range(k - 1, -1, -1):
            arr = stage(arr, 1 << j, 1 << k)
    t_s = arr[:, :B]
    dd_s = arr[:, B:2 * B]
    pk_s = jax.lax.bitcast_convert_type(arr[:, 2 * B:], jnp.int32)
    inv1023 = jnp.float32(1.0 / 1023.0)
    cd_s = [((pk_s >> 20) & 1023).astype(jnp.float32) * inv1023,
            ((pk_s >> 10) & 1023).astype(jnp.float32) * inv1023,
            (pk_s & 1023).astype(jnp.float32) * inv1023]
    P_s = jnp.concatenate([dd_s] + cd_s, axis=1)

    def csum(n_steps, x):
        for i in range(n_steps):
            s = 1 << i
            zpad = jnp.zeros((s, x.shape[1]), jnp.float32)
            x = x + jnp.concatenate([zpad, x[:_E - s]], axis=0)
        return x

    CS = csum(9, P_s)
    D = CS[:, :B]
    t_next = jnp.concatenate([t_s[1:], t_s[_E - 1:]], axis=0)
    x = (t_next - t_s) * D
    S = csum(9, x)
    Aexp = S + jnp.where(rows > 0, x, zero)
    wgt = jnp.where(rows < _E - 1, jnp.exp(-Aexp), zero)
    wc = wgt / jnp.maximum(D, 1e-6)
    outs = [jnp.sum(wc * CS[:, (c + 1) * B:(c + 2) * B], axis=0, keepdims=True)
            for c in range(3)]
    out_ref[:, :] = jnp.concatenate(outs, axis=0)


def _render(rays, params, M, B):
    return pl.pallas_call(
        _renderer_kernel,
        grid=(M // B,),
        in_specs=[
            pl.BlockSpec((6, B), lambda i: (0, i)),
            pl.BlockSpec((_N, 19), lambda i: (0, 0)),
        ],
        out_specs=pl.BlockSpec((3, B), lambda i: (0, i)),
        out_shape=jax.ShapeDtypeStruct((3, M), jnp.float32),
        compiler_params=pltpu.CompilerParams(
            dimension_semantics=("parallel",)),
    )(rays, params)


def kernel(means3D, rays_o, rays_d, rotations, scales, colors, opacities,
           K, R, t):
    M = rays_o.shape[0]
    B = 256
    rays = jnp.concatenate([rays_o.T, rays_d.T], axis=0)
    params = jnp.concatenate(
        [means3D, rotations.reshape(_N, 9), scales, colors, opacities],
        axis=1)
    out = _render(rays, params, M, B)
    return out.T


# speedup vs baseline: 78.1206x; 1.0222x over previous
import functools

import numpy as np
import jax
import jax.numpy as jnp
from jax.experimental import pallas as pl
from jax.experimental.pallas import tpu as pltpu

_N = 256
_E = 2 * _N

_STAGES = [(1 << j, 1 << k)
           for k in range(1, _E.bit_length())
           for j in range(k - 1, -1, -1)]
_SS = np.array([s for s, _ in _STAGES], dtype=np.int32)
_KK = np.array([k for _, k in _STAGES], dtype=np.int32)


def _renderer_kernel(rays_ref, par_ref, out_ref):
    B = rays_ref.shape[1]
    rows = jax.lax.broadcasted_iota(jnp.int32, (_E, 1), 0)

    o = [rays_ref[i:i + 1, :] for i in range(3)]
    d = [rays_ref[3 + i:4 + i, :] for i in range(3)]
    dninv = 1.0 / jnp.sqrt(d[0] * d[0] + d[1] * d[1] + d[2] * d[2])
    d = [di * dninv for di in d]

    mean = [par_ref[:, c:c + 1] for c in range(0, 3)]
    scl = [par_ref[:, 12 + c:13 + c] for c in range(3)]
    A = [[par_ref[:, 3 + 3 * j + x:4 + 3 * j + x] / scl[x]
          for j in range(3)] for x in range(3)]
    w = [-(A[x][0] * mean[0] + A[x][1] * mean[1] + A[x][2] * mean[2])
         for x in range(3)]
    op = par_ref[:, 18:19]
    cop = [par_ref[:, 15 + c:16 + c] * op for c in range(3)]

    u = [A[x][0] * o[0] + A[x][1] * o[1] + A[x][2] * o[2] + w[x]
         for x in range(3)]
    v = [A[x][0] * d[0] + A[x][1] * d[1] + A[x][2] * d[2] for x in range(3)]
    qa = v[0] * v[0] + v[1] * v[1] + v[2] * v[2]
    qb = 2.0 * (u[0] * v[0] + u[1] * v[1] + u[2] * v[2])
    qc = u[0] * u[0] + u[1] * u[1] + u[2] * u[2] - 1.0
    disc = qb * qb - 4.0 * qa * qc
    valid = disc >= 0.0
    sq = jnp.sqrt(jnp.maximum(disc, 0.0))
    t0 = (-qb - sq) / (2.0 * qa)
    t1 = (-qb + sq) / (2.0 * qa)
    hit0 = (t0 > 0.0) & valid
    hit1 = (t1 > 0.0) & valid
    t0 = jnp.where(hit0, t0, -1.0)
    t1 = jnp.where(hit1, t1, -1.0)

    zero = jnp.zeros((), jnp.float32)
    tcat = jnp.concatenate([t0, t1], axis=0)
    dd = jnp.concatenate([jnp.where(hit0, op, zero),
                          -jnp.where(hit1, op, zero)], axis=0)
    qc = [jnp.round(cop[c] * 1023.0).astype(jnp.int32) for c in range(3)]
    packi = (qc[0] << 20) | (qc[1] << 10) | qc[2]
    packf = jax.lax.bitcast_convert_type(packi, jnp.float32)
    pcd = jnp.concatenate([jnp.where(hit0, packf, zero),
                           jnp.where(hit1, packf, zero)], axis=0)
    arr = jnp.concatenate([tcat, dd, pcd], axis=1)

    sign_bit = jnp.int32(-2147483648)
    i0 = jnp.int32(0)

    def stage(a, s, kk):
        is_lo = (rows & s) == 0
        desc = (rows & kk) != 0
        want_min = jnp.logical_xor(is_lo, desc)
        zp = jnp.zeros((s, a.shape[1]), jnp.float32)
        a_dn = jnp.concatenate([a[s:], zp], axis=0)
        a_up = jnp.concatenate([zp, a[:_E - s]], axis=0)
        ap = jnp.where(is_lo, a_dn, a_up)
        t = a[:, :B]
        pk = ap[:, :B]
        flip = jnp.where(want_min, i0, sign_bit)
        d = jax.lax.bitcast_convert_type(t - pk, jnp.int32) ^ flip
        take = jax.lax.bitcast_convert_type(d, jnp.float32) > 0.0
        take3 = jnp.concatenate([take, take, take], axis=1)
        return jnp.where(take3, ap, a)

    for k in range(1, _E.bit_length()):
        for j in range(k - 1, -1, -1):
            arr = stage(arr, 1 << j, 1 << k)
    t_s = arr[:, :B]
    dd_s = arr[:, B:2 * B]
    pk_s = jax.lax.bitcast_convert_type(arr[:, 2 * B:], jnp.int32)
    inv1023 = jnp.float32(1.0 / 1023.0)
    cd_s = [((pk_s >> 20) & 1023).astype(jnp.float32) * inv1023,
            ((pk_s >> 10) & 1023).astype(jnp.float32) * inv1023,
            (pk_s & 1023).astype(jnp.float32) * inv1023]
    P_s = jnp.concatenate([dd_s] + cd_s, axis=1)

    def csum(n_steps, x):
        for i in range(n_steps):
            s = 1 << i
            zpad = jnp.zeros((s, x.shape[1]), jnp.float32)
            x = x + jnp.concatenate([zpad, x[:_E - s]], axis=0)
        return x

    CS = csum(9, P_s)
    D = CS[:, :B]
    t_next = jnp.concatenate([t_s[1:], t_s[_E - 1:]], axis=0)
    x = (t_next - t_s) * D
    S = csum(9, x)
    Aexp = S + jnp.where(rows > 0, x, zero)
    wgt = jnp.where(rows < _E - 1, jnp.exp(-Aexp), zero)
    wc = wgt / jnp.maximum(D, 1e-6)
    outs = [jnp.sum(wc * CS[:, (c + 1) * B:(c + 2) * B], axis=0, keepdims=True)
            for c in range(3)]
    out_ref[:, :] = jnp.concatenate(outs, axis=0)


def _render(rays, params, M, B):
    return pl.pallas_call(
        _renderer_kernel,
        grid=(M // B,),
        in_specs=[
            pl.BlockSpec((6, B), lambda i: (0, i)),
            pl.BlockSpec((_N, 19), lambda i: (0, 0)),
        ],
        out_specs=pl.BlockSpec((3, B), lambda i: (0, i)),
        out_shape=jax.ShapeDtypeStruct((3, M), jnp.float32),
        compiler_params=pltpu.CompilerParams(
            dimension_semantics=("parallel",)),
    )(rays, params)


def kernel(means3D, rays_o, rays_d, rotations, scales, colors, opacities,
           K, R, t):
    M = rays_o.shape[0]
    B = 256
    rays = jnp.concatenate([rays_o.T, rays_d.T], axis=0)
    params = jnp.concatenate(
        [means3D, rotations.reshape(_N, 9), scales, colors, opacities],
        axis=1)
    out = _render(rays, params, M, B)
    return out.T
